# baseline (speedup 1.0000x reference)
import functools

import jax
import jax.numpy as jnp
from jax import lax
from jax.experimental import pallas as pl
from jax.experimental.pallas import tpu as pltpu
from jax.experimental.pallas import tpu_sc as plsc

N_NODES = 10000
N_EDGES = 320000
IN_CH = 128
TIME_DIM = 64
OUT_CH = 128

NC, NS = 2, 16
NW = NC * NS
EPW = 10240
KC = 128
NCH = EPW // KC
NBUF = 2
GROUPS = NCH // NBUF
NPAD = 10240
HALF = NPAD // 2
SPT = HALF // NS
ACCR = HALF + NS
HISTN = 10256
EPT = NW * EPW // NS
NCHB = EPT // KC
GROUPSB = NCHB // NBUF
HCH = IN_CH // 2
ACCB = NPAD + NS
SPTB = NPAD // NS

TWO_PI = 6.283185307179586
INV_TWO_PI = 0.15915494309189535
S1, S2, S3, S4 = 0.9994501730582424, -0.16583842947680918, 0.00799857532016737, -0.0001477404380785219
C0, C1, C2, C3, C4 = 0.999971093218446, -0.49983759608563205, 0.04152230455017175, -0.0013441068677429546, 1.906521608688954e-05


def _pipeline_pass(load, wait_load, dloc, bufs, acc, gsems, ssems, groups):
    for bb in range(NBUF):
        load(bb, bufs[bb], gsems[bb])

    def pstep(gg, carry):
        jb = NBUF * gg
        for bb in range(NBUF):
            wait_load(jb + bb, bufs[bb], gsems[bb])
            pltpu.async_copy(bufs[bb], acc.at[dloc.at[jb + bb]], ssems[bb],
                             add=True)
        for bb in range(NBUF):
            pltpu.make_async_copy(bufs[bb], acc.at[dloc.at[jb + bb]],
                                  ssems[bb]).wait()

            @pl.when(gg < groups - 1)
            def _():
                load(jb + NBUF + bb, bufs[bb], gsems[bb])

        return carry

    lax.fori_loop(0, groups, pstep, 0)


def _localize(dstv, dloc, base, t):
    trash = HALF + t

    def cvt(j, carry):
        for kk in range(KC // 16):
            d = dstv[j, pl.ds(kk * 16, 16)]
            lo = d - base
            m = (lo >= 0) & (lo < HALF)
            dloc[j, pl.ds(kk * 16, 16)] = jnp.where(m, lo, trash)
        return carry

    lax.fori_loop(0, NCH, cvt, 0)


def _sca_body(x, src3, dst3, zeros, zeros1, outa, outd,
              srcv, dstv, dloc, r0, r1, hist, acc, spill,
              gs0, gs1, ss0, ss1, ws0, ws1):
    c = lax.axis_index("c")
    t = lax.axis_index("s")
    w = c * NS + t

    pltpu.sync_copy(src3.at[w], srcv)
    pltpu.sync_copy(dst3.at[w], dstv)
    pltpu.sync_copy(zeros1, hist)

    ones16 = jnp.ones((16,), jnp.float32)

    def hstep(j, carry):
        for kk in range(KC // 16):
            idx = dstv[j, pl.ds(kk * 16, 16)]
            plsc.addupdate_scatter(hist, [idx], ones16)
        return carry

    lax.fori_loop(0, NCH, hstep, 0)

    _localize(dstv, dloc, 0, t)
    pltpu.sync_copy(zeros, acc.at[pl.ds(t * SPT, SPT)])
    plsc.subcore_barrier()

    def g_load(j, r, sem):
        pltpu.async_copy(x.at[srcv.at[j]], r, sem)

    g_load(0, r0, gs0)
    g_load(1, r1, gs1)

    def pstep0(gg, carry):
        for bb, (r, gs, ss, ws) in enumerate(((r0, gs0, ss0, ws0),
                                              (r1, gs1, ss1, ws1))):
            j = 2 * gg + bb
            pltpu.make_async_copy(x.at[srcv.at[j]], r, gs).wait()
            pltpu.async_copy(r, acc.at[dloc.at[j]], ss, add=True)
            pltpu.async_copy(r, spill.at[w, pl.ds(j * KC, KC)], ws)
        for bb, (r, gs, ss, ws) in enumerate(((r0, gs0, ss0, ws0),
                                              (r1, gs1, ss1, ws1))):
            j = 2 * gg + bb
            pltpu.make_async_copy(r, acc.at[dloc.at[j]], ss).wait()
            pltpu.make_async_copy(r, spill.at[w, pl.ds(j * KC, KC)], ws).wait()

            @pl.when(gg < GROUPS - 1)
            def _():
                g_load(j + 2, r, gs)

        return carry

    lax.fori_loop(0, GROUPS, pstep0, 0)
    plsc.subcore_barrier()
    pltpu.sync_copy(acc.at[pl.ds(t * SPT, SPT)],
                    outa.at[c, pl.ds(t * SPT, SPT)])

    def l_load(j, r, sem):
        pltpu.async_copy(spill.at[w, pl.ds(j * KC, KC)], r, sem)

    def l_wait(j, r, sem):
        pltpu.make_async_copy(spill.at[w, pl.ds(j * KC, KC)], r, sem).wait()

    _localize(dstv, dloc, HALF, t)
    pltpu.sync_copy(zeros, acc.at[pl.ds(t * SPT, SPT)])
    plsc.subcore_barrier()
    _pipeline_pass(l_load, l_wait, dloc, (r0, r1), acc, (gs0, gs1), (ss0, ss1),
                   GROUPS)
    plsc.subcore_barrier()
    pltpu.sync_copy(acc.at[pl.ds(t * SPT, SPT)],
                    outa.at[c, pl.ds(HALF + t * SPT, SPT)])

    pltpu.sync_copy(hist.at[pl.ds(0, NPAD)], outd.at[c, t])


def _scb_body(tes, tec, dstb, zerosb, outb, dstv, r0, r1, acc,
              gs0, gs1, ss0, ss1):
    c = lax.axis_index("c")
    t = lax.axis_index("s")

    pltpu.sync_copy(dstb.at[t], dstv)

    def load(j, r, sem):
        @pl.when(c == 0)
        def _():
            pltpu.async_copy(tes.at[t, pl.ds(j * KC, KC)], r, sem)

        @pl.when(c == 1)
        def _():
            pltpu.async_copy(tec.at[t, pl.ds(j * KC, KC)], r, sem)

    def wait_load(j, r, sem):
        pltpu.make_async_copy(tes.at[t, pl.ds(j * KC, KC)], r, sem).wait()

    pltpu.sync_copy(zerosb, acc.at[pl.ds(t * SPTB, SPTB)])
    plsc.subcore_barrier()
    _pipeline_pass(load, wait_load, dstv, (r0, r1), acc,
                   (gs0, gs1), (ss0, ss1), GROUPSB)
    plsc.subcore_barrier()
    pltpu.sync_copy(acc.at[pl.ds(t * SPTB, SPTB)],
                    outb.at[c, pl.ds(t * SPTB, SPTB)])


_SC_MESH = dict(core_axis_name="c", subcore_axis_name="s",
                num_cores=NC, num_subcores=NS)
_IDX = lambda: pltpu.VMEM((NCH, KC), jnp.int32)
_ROWS = lambda: pltpu.VMEM((KC, IN_CH), jnp.float32)


@functools.cache
def _make_sca():
    return pl.kernel(
        _sca_body,
        out_type=(jax.ShapeDtypeStruct((NC, NPAD, IN_CH), jnp.float32),
                  jax.ShapeDtypeStruct((NC, NS, NPAD), jnp.float32)),
        mesh=plsc.VectorSubcoreMesh(**_SC_MESH),
        compiler_params=pltpu.CompilerParams(use_tc_tiling_on_sc=False, needs_layout_passes=False),
        scratch_types=[
            _IDX(), _IDX(), _IDX(),
            _ROWS(), _ROWS(),
            pltpu.VMEM((HISTN,), jnp.float32),
            pltpu.VMEM_SHARED((ACCR, IN_CH), jnp.float32),
            pltpu.HBM((NW, EPW, IN_CH), jnp.float32),
        ] + [pltpu.SemaphoreType.DMA] * 6,
    )


@functools.cache
def _make_scb():
    return pl.kernel(
        _scb_body,
        out_type=jax.ShapeDtypeStruct((NC, NPAD, HCH), jnp.float32),
        mesh=plsc.VectorSubcoreMesh(**_SC_MESH),
        compiler_params=pltpu.CompilerParams(use_tc_tiling_on_sc=False, needs_layout_passes=False),
        scratch_types=[
            pltpu.VMEM((NCHB, KC), jnp.int32),
            pltpu.VMEM((KC, HCH), jnp.float32),
            pltpu.VMEM((KC, HCH), jnp.float32),
            pltpu.VMEM_SHARED((ACCB, HCH), jnp.float32),
        ] + [pltpu.SemaphoreType.DMA] * 4,
    )


def _tenc_body(ts_ref, f_ref, out_ref):
    for i in range(ts_ref.shape[0]):
        tf = lax.dot_general(ts_ref[i:i + 1, :], f_ref[...],
                             (((0,), (0,)), ((), ())),
                             preferred_element_type=jnp.float32)
        u = tf * INV_TWO_PI
        r = u - jnp.round(u)
        th = r * TWO_PI
        z = th * th
        s = th * (S1 + z * (S2 + z * (S3 + z * S4)))
        co = C0 + z * (C1 + z * (C2 + z * (C3 + z * C4)))
        out_ref[i] = jnp.concatenate([s, co], axis=1)


def _fin_body(a0_ref, a1_ref, t0_ref, t1_ref, d_ref, w1_ref, w2_ref, b_ref, out_ref):
    a = a0_ref[...] + a1_ref[...]
    tt = jnp.concatenate([t0_ref[...], t1_ref[...]], axis=1)
    acc = jnp.dot(a, w1_ref[...], preferred_element_type=jnp.float32)
    acc += jnp.dot(tt, w2_ref[...], preferred_element_type=jnp.float32)
    acc += jnp.sum(d_ref[...], axis=0)[:, None] * b_ref[...]
    out_ref[...] = jnp.maximum(acc, 0.0)


def kernel(x, edge_index, edge_timestamps, freqs, W, b):
    src = edge_index[0].astype(jnp.int32)
    dst = edge_index[1].astype(jnp.int32)
    ts = edge_timestamps.astype(jnp.float32)

    epw0 = N_EDGES // NW
    padw = ((0, 0), (0, EPW - epw0))
    src3 = jnp.pad(src.reshape(NW, epw0), padw).reshape(NW, NCH, KC)
    fake = jnp.broadcast_to((NPAD + jnp.arange(NW, dtype=jnp.int32) % NS)[:, None],
                            (NW, EPW - epw0))
    dstp = jnp.concatenate([dst.reshape(NW, epw0), fake], axis=1)
    dst3 = dstp.reshape(NW, NCH, KC)
    dstb = dstp.reshape(NS, NCHB, KC)
    ts32 = jnp.pad(ts.reshape(NW, epw0), padw)
    zeros = jnp.zeros((SPT, IN_CH), jnp.float32)
    zeros1 = jnp.zeros((HISTN,), jnp.float32)
    zerosb = jnp.zeros((SPTB, HCH), jnp.float32)

    acca, deg = _make_sca()(x, src3, dst3, zeros, zeros1)

    tb = 2560
    rows = NW * EPW // tb
    te = pl.pallas_call(
        _tenc_body,
        grid=(rows // 8,),
        in_specs=[
            pl.BlockSpec((8, tb), lambda i: (i, 0)),
            pl.BlockSpec((1, TIME_DIM), lambda i: (0, 0)),
        ],
        out_specs=pl.BlockSpec((8, tb, 2 * TIME_DIM), lambda i: (i, 0, 0)),
        out_shape=jax.ShapeDtypeStruct((rows, tb, 2 * TIME_DIM), jnp.float32),
    )(ts32.reshape(rows, tb), freqs[None, :])
    te = te.reshape(NS, EPT, 2 * TIME_DIM)
    tes, tec = te[:, :, :HCH], te[:, :, HCH:]

    acct = _make_scb()(tes, tec, dstb, zerosb)

    nb = 1024
    deg2 = deg.reshape(NW, NPAD)
    out = pl.pallas_call(
        _fin_body,
        grid=(NPAD // nb,),
        in_specs=[
            pl.BlockSpec((nb, IN_CH), lambda i: (i, 0)),
            pl.BlockSpec((nb, IN_CH), lambda i: (i, 0)),
            pl.BlockSpec((nb, HCH), lambda i: (i, 0)),
            pl.BlockSpec((nb, HCH), lambda i: (i, 0)),
            pl.BlockSpec((NW, nb), lambda i: (0, i)),
            pl.BlockSpec((IN_CH, OUT_CH), lambda i: (0, 0)),
            pl.BlockSpec((2 * TIME_DIM, OUT_CH), lambda i: (0, 0)),
            pl.BlockSpec((1, OUT_CH), lambda i: (0, 0)),
        ],
        out_specs=pl.BlockSpec((nb, OUT_CH), lambda i: (i, 0)),
        out_shape=jax.ShapeDtypeStruct((NPAD, OUT_CH), jnp.float32),
    )(acca[0], acca[1], acct[0], acct[1], deg2, W[:IN_CH], W[IN_CH:], b[None, :])
    return out[:N_NODES]


# speedup vs baseline: 1.6456x; 1.6456x over previous
import functools

import jax
import jax.numpy as jnp
from jax import lax
from jax.experimental import pallas as pl
from jax.experimental.pallas import tpu as pltpu
from jax.experimental.pallas import tpu_sc as plsc

N_NODES = 10000
N_EDGES = 320000
IN_CH = 128
TIME_DIM = 64
OUT_CH = 128

NC, NS = 2, 16
NW = NC * NS
EPW = 10240
KC = 128
NCH = EPW // KC
NBUF = 2
GROUPS = NCH // NBUF
NPAD = 10240
HALF = NPAD // 2
SPT = HALF // NS
ACCR = HALF + NS
HISTN = 10256
EPT = NW * EPW // NS
NCHB = EPT // KC
GROUPSB = NCHB // NBUF
HCH = IN_CH // 2
ACCB = NPAD + NS
SPTB = NPAD // NS

TWO_PI = 6.283185307179586
INV_TWO_PI = 0.15915494309189535
S1, S2, S3, S4 = 0.9994501730582424, -0.16583842947680918, 0.00799857532016737, -0.0001477404380785219
C0, C1, C2, C3, C4 = 0.999971093218446, -0.49983759608563205, 0.04152230455017175, -0.0013441068677429546, 1.906521608688954e-05


def _pipeline_pass(load, wait_load, dloc, bufs, acc, gsems, ssems, groups):
    for bb in range(NBUF):
        load(bb, bufs[bb], gsems[bb])

    def pstep(gg, carry):
        jb = NBUF * gg
        for bb in range(NBUF):
            wait_load(jb + bb, bufs[bb], gsems[bb])
            pltpu.async_copy(bufs[bb], acc.at[dloc.at[jb + bb]], ssems[bb],
                             add=True)
        for bb in range(NBUF):
            pltpu.make_async_copy(bufs[bb], acc.at[dloc.at[jb + bb]],
                                  ssems[bb]).wait()

            @pl.when(gg < groups - 1)
            def _():
                load(jb + NBUF + bb, bufs[bb], gsems[bb])

        return carry

    lax.fori_loop(0, groups, pstep, 0)


def _localize(dstv, dloc, base, t):
    trash = HALF + t

    def cvt(j, carry):
        for kk in range(KC // 16):
            d = dstv[j, pl.ds(kk * 16, 16)]
            lo = d - base
            m = (lo >= 0) & (lo < HALF)
            dloc[j, pl.ds(kk * 16, 16)] = jnp.where(m, lo, trash)
        return carry

    lax.fori_loop(0, NCH, cvt, 0)


def _sca_body(x, src3, dst3, zeros, zeros1, outa, outd,
              srcv, dstv, dloc, r0, r1, hist, acc, spill,
              gs0, gs1, ss0, ss1, ws0, ws1):
    c = lax.axis_index("c")
    t = lax.axis_index("s")
    w = c * NS + t

    pltpu.sync_copy(src3.at[w], srcv)
    pltpu.sync_copy(dst3.at[w], dstv)
    pltpu.sync_copy(zeros1, hist)

    ones16 = jnp.ones((16,), jnp.float32)

    def hstep(j, carry):
        for kk in range(KC // 16):
            idx = dstv[j, pl.ds(kk * 16, 16)]
            plsc.addupdate_scatter(hist, [idx], ones16)
        return carry

    lax.fori_loop(0, NCH, hstep, 0)

    _localize(dstv, dloc, 0, t)
    pltpu.sync_copy(zeros, acc.at[pl.ds(t * SPT, SPT)])
    plsc.subcore_barrier()

    def g_load(j, r, sem):
        pltpu.async_copy(x.at[srcv.at[j]], r, sem)

    g_load(0, r0, gs0)
    g_load(1, r1, gs1)

    def pstep0(gg, carry):
        for bb, (r, gs, ss, ws) in enumerate(((r0, gs0, ss0, ws0),
                                              (r1, gs1, ss1, ws1))):
            j = 2 * gg + bb
            pltpu.make_async_copy(x.at[srcv.at[j]], r, gs).wait()
            pltpu.async_copy(r, acc.at[dloc.at[j]], ss, add=True)
            pltpu.async_copy(r, spill.at[w, pl.ds(j * KC, KC)], ws)
        for bb, (r, gs, ss, ws) in enumerate(((r0, gs0, ss0, ws0),
                                              (r1, gs1, ss1, ws1))):
            j = 2 * gg + bb
            pltpu.make_async_copy(r, acc.at[dloc.at[j]], ss).wait()
            pltpu.make_async_copy(r, spill.at[w, pl.ds(j * KC, KC)], ws).wait()

            @pl.when(gg < GROUPS - 1)
            def _():
                g_load(j + 2, r, gs)

        return carry

    lax.fori_loop(0, GROUPS, pstep0, 0)
    plsc.subcore_barrier()
    pltpu.sync_copy(acc.at[pl.ds(t * SPT, SPT)],
                    outa.at[c, pl.ds(t * SPT, SPT)])

    def l_load(j, r, sem):
        pltpu.async_copy(spill.at[w, pl.ds(j * KC, KC)], r, sem)

    def l_wait(j, r, sem):
        pltpu.make_async_copy(spill.at[w, pl.ds(j * KC, KC)], r, sem).wait()

    _localize(dstv, dloc, HALF, t)
    pltpu.sync_copy(zeros, acc.at[pl.ds(t * SPT, SPT)])
    plsc.subcore_barrier()
    _pipeline_pass(l_load, l_wait, dloc, (r0, r1), acc, (gs0, gs1), (ss0, ss1),
                   GROUPS)
    plsc.subcore_barrier()
    pltpu.sync_copy(acc.at[pl.ds(t * SPT, SPT)],
                    outa.at[c, pl.ds(HALF + t * SPT, SPT)])

    pltpu.sync_copy(hist.at[pl.ds(0, NPAD)], outd.at[c, t])


def _scb_body(tes, tec, dstb, zerosb, outb, dstv, r0, r1, acc,
              gs0, gs1, ss0, ss1):
    c = lax.axis_index("c")
    t = lax.axis_index("s")

    pltpu.sync_copy(dstb.at[t], dstv)

    def load(j, r, sem):
        @pl.when(c == 0)
        def _():
            pltpu.async_copy(tes.at[t, pl.ds(j * KC, KC)], r, sem)

        @pl.when(c == 1)
        def _():
            pltpu.async_copy(tec.at[t, pl.ds(j * KC, KC)], r, sem)

    def wait_load(j, r, sem):
        pltpu.make_async_copy(tes.at[t, pl.ds(j * KC, KC)], r, sem).wait()

    pltpu.sync_copy(zerosb, acc.at[pl.ds(t * SPTB, SPTB)])
    plsc.subcore_barrier()
    _pipeline_pass(load, wait_load, dstv, (r0, r1), acc,
                   (gs0, gs1), (ss0, ss1), GROUPSB)
    plsc.subcore_barrier()
    pltpu.sync_copy(acc.at[pl.ds(t * SPTB, SPTB)],
                    outb.at[c, pl.ds(t * SPTB, SPTB)])


_SC_MESH = dict(core_axis_name="c", subcore_axis_name="s",
                num_cores=NC, num_subcores=NS)
_IDX = lambda: pltpu.VMEM((NCH, KC), jnp.int32)
_ROWS = lambda: pltpu.VMEM((KC, IN_CH), jnp.float32)


@functools.cache
def _make_sca():
    return pl.kernel(
        _sca_body,
        out_type=(jax.ShapeDtypeStruct((NC, NPAD, IN_CH), jnp.float32),
                  jax.ShapeDtypeStruct((NC, NS, NPAD), jnp.float32)),
        mesh=plsc.VectorSubcoreMesh(**_SC_MESH),
        compiler_params=pltpu.CompilerParams(use_tc_tiling_on_sc=False, needs_layout_passes=False),
        scratch_types=[
            _IDX(), _IDX(), _IDX(),
            _ROWS(), _ROWS(),
            pltpu.VMEM((HISTN,), jnp.float32),
            pltpu.VMEM_SHARED((ACCR, IN_CH), jnp.float32),
            pltpu.HBM((NW, EPW, IN_CH), jnp.float32),
        ] + [pltpu.SemaphoreType.DMA] * 6,
    )


@functools.cache
def _make_scb():
    return pl.kernel(
        _scb_body,
        out_type=jax.ShapeDtypeStruct((NC, NPAD, HCH), jnp.float32),
        mesh=plsc.VectorSubcoreMesh(**_SC_MESH),
        compiler_params=pltpu.CompilerParams(use_tc_tiling_on_sc=False, needs_layout_passes=False),
        scratch_types=[
            pltpu.VMEM((NCHB, KC), jnp.int32),
            pltpu.VMEM((KC, HCH), jnp.float32),
            pltpu.VMEM((KC, HCH), jnp.float32),
            pltpu.VMEM_SHARED((ACCB, HCH), jnp.float32),
        ] + [pltpu.SemaphoreType.DMA] * 4,
    )


def _sincos(tf):
    u = tf * INV_TWO_PI
    r = u - jnp.round(u)
    th = r * TWO_PI
    z = th * th
    s = th * (S1 + z * (S2 + z * (S3 + z * S4)))
    co = C0 + z * (C1 + z * (C2 + z * (C3 + z * C4)))
    return s, co


def _tenc_body(tse_ref, tso_ref, f_ref, outs_ref, outc_ref):
    for i in range(tse_ref.shape[0]):
        dn = (((0,), (0,)), ((), ()))
        tfe = lax.dot_general(tse_ref[i:i + 1, :], f_ref[...], dn,
                              preferred_element_type=jnp.float32)
        tfo = lax.dot_general(tso_ref[i:i + 1, :], f_ref[...], dn,
                              preferred_element_type=jnp.float32)
        se, ce = _sincos(tfe)
        so, co = _sincos(tfo)
        outs_ref[i] = jnp.concatenate([se, so], axis=1)
        outc_ref[i] = jnp.concatenate([ce, co], axis=1)


def _fin_body(a0_ref, a1_ref, t0_ref, t1_ref, d_ref, w1_ref, w2_ref, b_ref, out_ref):
    a = a0_ref[...] + a1_ref[...]
    tt = jnp.concatenate([t0_ref[...], t1_ref[...]], axis=1)
    acc = jnp.dot(a, w1_ref[...], preferred_element_type=jnp.float32)
    acc += jnp.dot(tt, w2_ref[...], preferred_element_type=jnp.float32)
    acc += jnp.sum(d_ref[...], axis=0)[:, None] * b_ref[...]
    out_ref[...] = jnp.maximum(acc, 0.0)


def kernel(x, edge_index, edge_timestamps, freqs, W, b):
    src = edge_index[0].astype(jnp.int32)
    dst = edge_index[1].astype(jnp.int32)
    ts = edge_timestamps.astype(jnp.float32)

    epw0 = N_EDGES // NW
    padw = ((0, 0), (0, EPW - epw0))
    src3 = jnp.pad(src.reshape(NW, epw0), padw).reshape(NW, NCH, KC)
    fake = jnp.broadcast_to((NPAD + jnp.arange(NW, dtype=jnp.int32) % NS)[:, None],
                            (NW, EPW - epw0))
    dstp = jnp.concatenate([dst.reshape(NW, epw0), fake], axis=1)
    dst3 = dstp.reshape(NW, NCH, KC)
    dstb = dstp.reshape(NS, NCHB, KC)
    ts32 = jnp.pad(ts.reshape(NW, epw0), padw)
    zeros = jnp.zeros((SPT, IN_CH), jnp.float32)
    zeros1 = jnp.zeros((HISTN,), jnp.float32)
    zerosb = jnp.zeros((SPTB, HCH), jnp.float32)

    acca, deg = _make_sca()(x, src3, dst3, zeros, zeros1)

    tb = 1280
    rows = NW * EPW // (2 * tb)
    tsf = ts32.reshape(-1)
    tse2 = tsf[0::2].reshape(rows, tb)
    tso2 = tsf[1::2].reshape(rows, tb)
    tes, tec = pl.pallas_call(
        _tenc_body,
        grid=(rows // 8,),
        in_specs=[
            pl.BlockSpec((8, tb), lambda i: (i, 0)),
            pl.BlockSpec((8, tb), lambda i: (i, 0)),
            pl.BlockSpec((1, TIME_DIM), lambda i: (0, 0)),
        ],
        out_specs=[
            pl.BlockSpec((8, tb, 2 * TIME_DIM), lambda i: (i, 0, 0)),
            pl.BlockSpec((8, tb, 2 * TIME_DIM), lambda i: (i, 0, 0)),
        ],
        out_shape=[
            jax.ShapeDtypeStruct((rows, tb, 2 * TIME_DIM), jnp.float32),
            jax.ShapeDtypeStruct((rows, tb, 2 * TIME_DIM), jnp.float32),
        ],
    )(tse2, tso2, freqs[None, :])
    tes = tes.reshape(NS, EPT, HCH)
    tec = tec.reshape(NS, EPT, HCH)

    zerosb = zerosb + deg[0, 0, 0] * 0.0
    acct = _make_scb()(tes, tec, dstb, zerosb)

    nb = 1024
    deg2 = deg.reshape(NW, NPAD)
    out = pl.pallas_call(
        _fin_body,
        grid=(NPAD // nb,),
        in_specs=[
            pl.BlockSpec((nb, IN_CH), lambda i: (i, 0)),
            pl.BlockSpec((nb, IN_CH), lambda i: (i, 0)),
            pl.BlockSpec((nb, HCH), lambda i: (i, 0)),
            pl.BlockSpec((nb, HCH), lambda i: (i, 0)),
            pl.BlockSpec((NW, nb), lambda i: (0, i)),
            pl.BlockSpec((IN_CH, OUT_CH), lambda i: (0, 0)),
            pl.BlockSpec((2 * TIME_DIM, OUT_CH), lambda i: (0, 0)),
            pl.BlockSpec((1, OUT_CH), lambda i: (0, 0)),
        ],
        out_specs=pl.BlockSpec((nb, OUT_CH), lambda i: (i, 0)),
        out_shape=jax.ShapeDtypeStruct((NPAD, OUT_CH), jnp.float32),
    )(acca[0], acca[1], acct[0], acct[1], deg2, W[:IN_CH], W[IN_CH:], b[None, :])
    return out[:N_NODES]
